# Initial kernel scaffold; baseline (speedup 1.0000x reference)
import jax
import jax.numpy as jnp
from jax.experimental import pallas as pl


def kernel(x, emb, W_g, W1, b1, W2, b2, W_fc, b_fc):
    raise NotImplementedError("write your pallas kernel here")


# speedup vs baseline: 7.5212x; 7.5212x over previous
import functools

import jax
import jax.numpy as jnp
from jax import lax
from jax.experimental import pallas as pl
from jax.experimental.pallas import tpu as pltpu
from jax.experimental.pallas import tpu_sc as plsc

_B = 4096
_L = 200
_D = 128
_E = 8
_FFN = 512
_OUT = 2

_NC = 2
_NS = 16
_NW = _NC * _NS
_TPW = _B // _NW
_C0 = 104
_C1 = 96


def _pool_body(x_hbm, emb_hbm, out_hbm, ids_v, rows_a, rows_b, out_v,
               sem_a, sem_b):
    wid = lax.axis_index("s") * _NC + lax.axis_index("c")
    tok0 = wid * _TPW
    pltpu.sync_copy(x_hbm.at[pl.ds(tok0 * _L, _TPW * _L)], ids_v)

    def acc_rows(rows_ref, n, accs):
        def rbody(r, accs):
            return tuple(accs[d] + rows_ref[r, pl.ds(d * 16, 16)]
                         for d in range(8))
        return lax.fori_loop(0, n, rbody, accs)

    def token_body(i, carry):
        base = i * _L
        cp_a = pltpu.async_copy(emb_hbm.at[ids_v.at[pl.ds(base, _C0)]],
                                rows_a, sem_a)
        cp_b = pltpu.async_copy(emb_hbm.at[ids_v.at[pl.ds(base + _C0, _C1)]],
                                rows_b, sem_b)
        cp_a.wait()
        accs = tuple(jnp.zeros((16,), jnp.float32) for _ in range(8))
        accs = acc_rows(rows_a, _C0, accs)
        cp_b.wait()
        accs = acc_rows(rows_b, _C1, accs)
        scale = jnp.float32(1.0 / _L)
        for d in range(8):
            out_v[i, pl.ds(d * 16, 16)] = accs[d] * scale
        return carry

    lax.fori_loop(0, _TPW, token_body, 0)
    pltpu.sync_copy(out_v, out_hbm.at[pl.ds(tok0, _TPW)])


@functools.cache
def _pool():
    return pl.kernel(
        _pool_body,
        out_type=jax.ShapeDtypeStruct((_B, _D), jnp.float32),
        mesh=plsc.VectorSubcoreMesh(core_axis_name="c", subcore_axis_name="s",
                                    num_cores=_NC, num_subcores=_NS),
        scratch_types=[
            pltpu.VMEM((_TPW * _L,), jnp.int32),
            pltpu.VMEM((_C0, _D), jnp.float32),
            pltpu.VMEM((_C1, _D), jnp.float32),
            pltpu.VMEM((_TPW, _D), jnp.float32),
            pltpu.SemaphoreType.DMA,
            pltpu.SemaphoreType.DMA,
        ],
    )


_BLK = 1024


def _moe_body(emb_ref, wg_ref, w1_ref, b1_ref, w2_ref, b2_ref, wfc_ref,
              bfc_ref, out_ref):
    e = emb_ref[...]
    logits = jnp.dot(e, wg_ref[...], preferred_element_type=jnp.float32)
    m = jnp.max(logits, axis=-1, keepdims=True)
    gate = 1.0 / jnp.sum(jnp.exp(logits - m), axis=-1, keepdims=True)
    lane = lax.broadcasted_iota(jnp.int32, logits.shape, 1)
    top1 = jnp.min(jnp.where(logits == m, lane, _E), axis=-1, keepdims=True)
    wfc = wfc_ref[...]
    b2fc = jnp.dot(b2_ref[...], wfc, preferred_element_type=jnp.float32)
    acc = jnp.zeros((e.shape[0], _OUT), jnp.float32)
    for ei in range(_E):
        h = jnp.maximum(
            jnp.dot(e, w1_ref[ei], preferred_element_type=jnp.float32)
            + b1_ref[ei], 0.0)
        w2fc = jnp.dot(w2_ref[ei], wfc, preferred_element_type=jnp.float32)
        t = jnp.dot(h, w2fc, preferred_element_type=jnp.float32) \
            + b2fc[ei:ei + 1, :]
        acc = acc + jnp.where(top1 == ei, t, 0.0)
    y = gate * acc + bfc_ref[...]
    my = jnp.max(y, axis=-1, keepdims=True)
    lse = my + jnp.log(jnp.sum(jnp.exp(y - my), axis=-1, keepdims=True))
    out_ref[...] = y - lse


def _moe(embedded, W_g, W1, b1, W2, b2, W_fc, b_fc, interpret=False):
    return pl.pallas_call(
        _moe_body,
        grid=(_B // _BLK,),
        in_specs=[
            pl.BlockSpec((_BLK, _D), lambda i: (i, 0)),
            pl.BlockSpec((_D, _E), lambda i: (0, 0)),
            pl.BlockSpec((_E, _D, _FFN), lambda i: (0, 0, 0)),
            pl.BlockSpec((_E, 1, _FFN), lambda i: (0, 0, 0)),
            pl.BlockSpec((_E, _FFN, _D), lambda i: (0, 0, 0)),
            pl.BlockSpec((_E, _D), lambda i: (0, 0)),
            pl.BlockSpec((_D, _OUT), lambda i: (0, 0)),
            pl.BlockSpec((1, _OUT), lambda i: (0, 0)),
        ],
        out_specs=pl.BlockSpec((_BLK, _OUT), lambda i: (i, 0)),
        out_shape=jax.ShapeDtypeStruct((_B, _OUT), jnp.float32),
        compiler_params=pltpu.CompilerParams(
            dimension_semantics=("arbitrary",)),
        interpret=interpret,
    )(embedded, W_g, W1, b1.reshape(_E, 1, _FFN), W2, b2, W_fc,
      b_fc.reshape(1, _OUT))


def kernel(x, emb, W_g, W1, b1, W2, b2, W_fc, b_fc):
    x_flat = x.reshape(-1).astype(jnp.int32)
    embedded = _pool()(x_flat, emb)
    return _moe(embedded, W_g, W1, b1, W2, b2, W_fc, b_fc)


# speedup vs baseline: 12.1588x; 1.6166x over previous
import functools

import jax
import jax.numpy as jnp
from jax import lax
from jax.experimental import pallas as pl
from jax.experimental.pallas import tpu as pltpu
from jax.experimental.pallas import tpu_sc as plsc

_B = 4096
_L = 200
_D = 128
_E = 8
_FFN = 512
_OUT = 2

_NC = 2
_NS = 16
_NW = _NC * _NS
_TPW = _B // _NW
_C0 = 104
_C1 = 96


_NBUF = 2


def _pool_body(x_hbm, emb_hbm, out_hbm, ids_v, rows_a, rows_b, out_v,
               sem_a, sem_b):
    wid = lax.axis_index("s") * _NC + lax.axis_index("c")
    tok0 = wid * _TPW
    pltpu.sync_copy(x_hbm.at[pl.ds(tok0 * _L, _TPW * _L)], ids_v)

    def copies(i, s):
        base = i * _L
        return (
            pltpu.make_async_copy(emb_hbm.at[ids_v.at[pl.ds(base, _C0)]],
                                  rows_a[s], sem_a[s]),
            pltpu.make_async_copy(
                emb_hbm.at[ids_v.at[pl.ds(base + _C0, _C1)]],
                rows_b[s], sem_b[s]),
        )

    def issue(i, s):
        ca, cb = copies(i, s)
        ca.start()
        cb.start()

    def acc_rows(rows_ref, n, accs):
        def rbody(r, accs):
            for u in range(4):
                accs = tuple(accs[d] + rows_ref[r * 4 + u, pl.ds(d * 16, 16)]
                             for d in range(8))
            return accs
        return lax.fori_loop(0, n // 4, rbody, accs)

    for s in range(_NBUF):
        issue(s, s)

    def token_body(j, carry):
        for s in range(_NBUF):
            i = j * _NBUF + s
            ca, cb = copies(i, s)
            ca.wait()
            accs = tuple(jnp.zeros((16,), jnp.float32) for _ in range(8))
            accs = acc_rows(rows_a[s], _C0, accs)
            cb.wait()
            accs = acc_rows(rows_b[s], _C1, accs)
            scale = jnp.float32(1.0 / _L)
            for d in range(8):
                out_v[i, pl.ds(d * 16, 16)] = accs[d] * scale

            @pl.when(i + _NBUF < _TPW)
            def _():
                issue(i + _NBUF, s)
        return carry

    lax.fori_loop(0, _TPW // _NBUF, token_body, 0)
    pltpu.sync_copy(out_v, out_hbm.at[pl.ds(tok0, _TPW)])


@functools.cache
def _pool():
    return pl.kernel(
        _pool_body,
        out_type=jax.ShapeDtypeStruct((_B, _D), jnp.float32),
        mesh=plsc.VectorSubcoreMesh(core_axis_name="c", subcore_axis_name="s",
                                    num_cores=_NC, num_subcores=_NS),
        scratch_types=[
            pltpu.VMEM((_TPW * _L,), jnp.int32),
            [pltpu.VMEM((_C0, _D), jnp.float32) for _ in range(_NBUF)],
            [pltpu.VMEM((_C1, _D), jnp.float32) for _ in range(_NBUF)],
            pltpu.VMEM((_TPW, _D), jnp.float32),
            [pltpu.SemaphoreType.DMA for _ in range(_NBUF)],
            [pltpu.SemaphoreType.DMA for _ in range(_NBUF)],
        ],
    )


_BLK = 1024


def _moe_body(emb_ref, wg_ref, w1_ref, b1_ref, w2_ref, b2_ref, wfc_ref,
              bfc_ref, out_ref):
    e = emb_ref[...]
    logits = jnp.dot(e, wg_ref[...], preferred_element_type=jnp.float32)
    m = jnp.max(logits, axis=-1, keepdims=True)
    gate = 1.0 / jnp.sum(jnp.exp(logits - m), axis=-1, keepdims=True)
    lane = lax.broadcasted_iota(jnp.int32, logits.shape, 1)
    top1 = jnp.min(jnp.where(logits == m, lane, _E), axis=-1, keepdims=True)
    wfc = wfc_ref[...]
    b2fc = jnp.dot(b2_ref[...], wfc, preferred_element_type=jnp.float32)
    acc = jnp.zeros((e.shape[0], _OUT), jnp.float32)
    for ei in range(_E):
        h = jnp.maximum(
            jnp.dot(e, w1_ref[ei], preferred_element_type=jnp.float32)
            + b1_ref[ei], 0.0)
        w2fc = jnp.dot(w2_ref[ei], wfc, preferred_element_type=jnp.float32)
        t = jnp.dot(h, w2fc, preferred_element_type=jnp.float32) \
            + b2fc[ei:ei + 1, :]
        acc = acc + jnp.where(top1 == ei, t, 0.0)
    y = gate * acc + bfc_ref[...]
    my = jnp.max(y, axis=-1, keepdims=True)
    lse = my + jnp.log(jnp.sum(jnp.exp(y - my), axis=-1, keepdims=True))
    out_ref[...] = y - lse


def _moe(embedded, W_g, W1, b1, W2, b2, W_fc, b_fc, interpret=False):
    return pl.pallas_call(
        _moe_body,
        grid=(_B // _BLK,),
        in_specs=[
            pl.BlockSpec((_BLK, _D), lambda i: (i, 0)),
            pl.BlockSpec((_D, _E), lambda i: (0, 0)),
            pl.BlockSpec((_E, _D, _FFN), lambda i: (0, 0, 0)),
            pl.BlockSpec((_E, 1, _FFN), lambda i: (0, 0, 0)),
            pl.BlockSpec((_E, _FFN, _D), lambda i: (0, 0, 0)),
            pl.BlockSpec((_E, _D), lambda i: (0, 0)),
            pl.BlockSpec((_D, _OUT), lambda i: (0, 0)),
            pl.BlockSpec((1, _OUT), lambda i: (0, 0)),
        ],
        out_specs=pl.BlockSpec((_BLK, _OUT), lambda i: (i, 0)),
        out_shape=jax.ShapeDtypeStruct((_B, _OUT), jnp.float32),
        compiler_params=pltpu.CompilerParams(
            dimension_semantics=("arbitrary",)),
        interpret=interpret,
    )(embedded, W_g, W1, b1.reshape(_E, 1, _FFN), W2, b2, W_fc,
      b_fc.reshape(1, _OUT))


def kernel(x, emb, W_g, W1, b1, W2, b2, W_fc, b_fc):
    x_flat = x.reshape(-1).astype(jnp.int32)
    embedded = _pool()(x_flat, emb)
    return _moe(embedded, W_g, W1, b1, W2, b2, W_fc, b_fc)


# speedup vs baseline: 14.1784x; 1.1661x over previous
import functools

import jax
import jax.numpy as jnp
from jax import lax
from jax.experimental import pallas as pl
from jax.experimental.pallas import tpu as pltpu
from jax.experimental.pallas import tpu_sc as plsc

_B = 4096
_L = 200
_D = 128
_E = 8
_FFN = 512
_OUT = 2

_NC = 2
_NS = 16
_NW = _NC * _NS
_TPW = _B // _NW
_C0 = 104
_C1 = 96


_NBUF = 3


def _pool_body(x_hbm, emb_hbm, out_hbm, ids_v, rows_a, rows_b, out_v,
               sem_a, sem_b):
    wid = lax.axis_index("s") * _NC + lax.axis_index("c")
    tok0 = wid * _TPW
    pltpu.sync_copy(x_hbm.at[pl.ds(tok0 * _L, _TPW * _L)], ids_v)

    def copies(i, s):
        base = i * _L
        return (
            pltpu.make_async_copy(emb_hbm.at[ids_v.at[pl.ds(base, _C0)]],
                                  rows_a[s], sem_a[s]),
            pltpu.make_async_copy(
                emb_hbm.at[ids_v.at[pl.ds(base + _C0, _C1)]],
                rows_b[s], sem_b[s]),
        )

    def issue(i, s):
        ca, cb = copies(i, s)
        ca.start()
        cb.start()

    def acc_rows(rows_ref, n, accs):
        def rbody(r, accs):
            for u in range(4):
                accs = tuple(accs[d] + rows_ref[r * 4 + u, pl.ds(d * 16, 16)]
                             for d in range(8))
            return accs
        return lax.fori_loop(0, n // 4, rbody, accs)

    for s in range(_NBUF):
        issue(s, s)

    def consume(i, s):
        ca, cb = copies(i, s)
        ca.wait()
        accs = tuple(jnp.zeros((16,), jnp.float32) for _ in range(8))
        accs = acc_rows(rows_a[s], _C0, accs)
        cb.wait()
        accs = acc_rows(rows_b[s], _C1, accs)
        scale = jnp.float32(1.0 / _L)
        for d in range(8):
            out_v[i, pl.ds(d * 16, 16)] = accs[d] * scale

    def token_body(j, carry):
        for s in range(_NBUF):
            i = j * _NBUF + s
            consume(i, s)

            @pl.when(i + _NBUF < _TPW)
            def _():
                issue(i + _NBUF, s)
        return carry

    _full = _TPW // _NBUF
    lax.fori_loop(0, _full, token_body, 0)
    for s in range(_TPW % _NBUF):
        consume(_full * _NBUF + s, s)
    pltpu.sync_copy(out_v, out_hbm.at[pl.ds(tok0, _TPW)])


@functools.cache
def _pool():
    return pl.kernel(
        _pool_body,
        out_type=jax.ShapeDtypeStruct((_B, _D), jnp.float32),
        mesh=plsc.VectorSubcoreMesh(core_axis_name="c", subcore_axis_name="s",
                                    num_cores=_NC, num_subcores=_NS),
        scratch_types=[
            pltpu.VMEM((_TPW * _L,), jnp.int32),
            [pltpu.VMEM((_C0, _D), jnp.float32) for _ in range(_NBUF)],
            [pltpu.VMEM((_C1, _D), jnp.float32) for _ in range(_NBUF)],
            pltpu.VMEM((_TPW, _D), jnp.float32),
            [pltpu.SemaphoreType.DMA for _ in range(_NBUF)],
            [pltpu.SemaphoreType.DMA for _ in range(_NBUF)],
        ],
    )


_BLK = 1024


def _moe_body(emb_ref, wg_ref, w1_ref, b1_ref, w2_ref, b2_ref, wfc_ref,
              bfc_ref, out_ref):
    e = emb_ref[...]
    logits = jnp.dot(e, wg_ref[...], preferred_element_type=jnp.float32)
    m = jnp.max(logits, axis=-1, keepdims=True)
    gate = 1.0 / jnp.sum(jnp.exp(logits - m), axis=-1, keepdims=True)
    lane = lax.broadcasted_iota(jnp.int32, logits.shape, 1)
    top1 = jnp.min(jnp.where(logits == m, lane, _E), axis=-1, keepdims=True)
    wfc = wfc_ref[...]
    b2fc = jnp.dot(b2_ref[...], wfc, preferred_element_type=jnp.float32)
    e16 = e.astype(jnp.bfloat16)
    acc = jnp.zeros((e.shape[0], _OUT), jnp.float32)
    for ei in range(_E):
        h = jnp.maximum(
            jnp.dot(e16, w1_ref[ei], preferred_element_type=jnp.float32)
            + b1_ref[ei], 0.0)
        w2fc = jnp.dot(w2_ref[ei], wfc, preferred_element_type=jnp.float32)
        t = jnp.dot(h.astype(jnp.bfloat16), w2fc.astype(jnp.bfloat16),
                    preferred_element_type=jnp.float32) \
            + b2fc[ei:ei + 1, :]
        acc = acc + jnp.where(top1 == ei, t, 0.0)
    y = gate * acc + bfc_ref[...]
    my = jnp.max(y, axis=-1, keepdims=True)
    lse = my + jnp.log(jnp.sum(jnp.exp(y - my), axis=-1, keepdims=True))
    out_ref[...] = y - lse


def _moe(embedded, W_g, W1, b1, W2, b2, W_fc, b_fc, interpret=False):
    return pl.pallas_call(
        _moe_body,
        grid=(_B // _BLK,),
        in_specs=[
            pl.BlockSpec((_BLK, _D), lambda i: (i, 0)),
            pl.BlockSpec((_D, _E), lambda i: (0, 0)),
            pl.BlockSpec((_E, _D, _FFN), lambda i: (0, 0, 0)),
            pl.BlockSpec((_E, 1, _FFN), lambda i: (0, 0, 0)),
            pl.BlockSpec((_E, _FFN, _D), lambda i: (0, 0, 0)),
            pl.BlockSpec((_E, _D), lambda i: (0, 0)),
            pl.BlockSpec((_D, _OUT), lambda i: (0, 0)),
            pl.BlockSpec((1, _OUT), lambda i: (0, 0)),
        ],
        out_specs=pl.BlockSpec((_BLK, _OUT), lambda i: (i, 0)),
        out_shape=jax.ShapeDtypeStruct((_B, _OUT), jnp.float32),
        compiler_params=pltpu.CompilerParams(
            dimension_semantics=("arbitrary",)),
        interpret=interpret,
    )(embedded, W_g, W1.astype(jnp.bfloat16), b1.reshape(_E, 1, _FFN), W2,
      b2, W_fc, b_fc.reshape(1, _OUT))


def kernel(x, emb, W_g, W1, b1, W2, b2, W_fc, b_fc):
    x_flat = x.reshape(-1).astype(jnp.int32)
    embedded = _pool()(x_flat, emb)
    return _moe(embedded, W_g, W1, b1, W2, b2, W_fc, b_fc)


# speedup vs baseline: 14.1837x; 1.0004x over previous
import functools

import jax
import jax.numpy as jnp
from jax import lax
from jax.experimental import pallas as pl
from jax.experimental.pallas import tpu as pltpu
from jax.experimental.pallas import tpu_sc as plsc

_B = 4096
_L = 200
_D = 128
_E = 8
_FFN = 512
_OUT = 2

_NC = 2
_NS = 16
_NW = _NC * _NS
_TPW = _B // _NW
_C0 = 104
_C1 = 96


_NBUF = 3


def _pool_body(x_hbm, emb_hbm, out_hbm, ids_v, rows_a, rows_b, out_v,
               sem_a, sem_b):
    wid = lax.axis_index("s") * _NC + lax.axis_index("c")
    tok0 = wid * _TPW
    pltpu.sync_copy(x_hbm.at[pl.ds(tok0 * _L, _TPW * _L)], ids_v)

    def copies(i, s):
        base = i * _L
        return (
            pltpu.make_async_copy(emb_hbm.at[ids_v.at[pl.ds(base, _C0)]],
                                  rows_a[s], sem_a[s]),
            pltpu.make_async_copy(
                emb_hbm.at[ids_v.at[pl.ds(base + _C0, _C1)]],
                rows_b[s], sem_b[s]),
        )

    def issue(i, s):
        ca, cb = copies(i, s)
        ca.start()
        cb.start()

    def acc_rows(rows_ref, n, accs):
        def rbody(r, accs):
            for u in range(4):
                accs = tuple(accs[d] + rows_ref[r * 4 + u, pl.ds(d * 16, 16)]
                             for d in range(8))
            return accs
        return lax.fori_loop(0, n // 4, rbody, accs)

    for s in range(_NBUF):
        issue(s, s)

    def consume(i, s):
        ca, cb = copies(i, s)
        ca.wait()
        accs = tuple(jnp.zeros((16,), jnp.float32) for _ in range(8))
        accs = acc_rows(rows_a[s], _C0, accs)
        cb.wait()
        accs = acc_rows(rows_b[s], _C1, accs)
        scale = jnp.float32(1.0 / _L)
        for d in range(8):
            out_v[i, pl.ds(d * 16, 16)] = accs[d] * scale

    def token_body(j, carry):
        for s in range(_NBUF):
            i = j * _NBUF + s
            consume(i, s)

            @pl.when(i + _NBUF < _TPW)
            def _():
                issue(i + _NBUF, s)
        return carry

    _full = _TPW // _NBUF
    lax.fori_loop(0, _full, token_body, 0)
    for s in range(_TPW % _NBUF):
        consume(_full * _NBUF + s, s)
    pltpu.sync_copy(out_v, out_hbm.at[pl.ds(tok0, _TPW)])


@functools.cache
def _pool():
    return pl.kernel(
        _pool_body,
        out_type=jax.ShapeDtypeStruct((_B, _D), jnp.float32),
        mesh=plsc.VectorSubcoreMesh(core_axis_name="c", subcore_axis_name="s",
                                    num_cores=_NC, num_subcores=_NS),
        scratch_types=[
            pltpu.VMEM((_TPW * _L,), jnp.int32),
            [pltpu.VMEM((_C0, _D), jnp.float32) for _ in range(_NBUF)],
            [pltpu.VMEM((_C1, _D), jnp.float32) for _ in range(_NBUF)],
            pltpu.VMEM((_TPW, _D), jnp.float32),
            [pltpu.SemaphoreType.DMA for _ in range(_NBUF)],
            [pltpu.SemaphoreType.DMA for _ in range(_NBUF)],
        ],
    )


_BLK = 1024


def _moe_body(emb_ref, wg_ref, w1_ref, b1_ref, w2_ref, b2_ref, wfc_ref,
              bfc_ref, out_ref):
    e = emb_ref[...]
    logits = jnp.dot(e, wg_ref[...], preferred_element_type=jnp.float32)
    m = jnp.max(logits, axis=-1, keepdims=True)
    gate = 1.0 / jnp.sum(jnp.exp(logits - m), axis=-1, keepdims=True)
    lane = lax.broadcasted_iota(jnp.int32, logits.shape, 1)
    top1 = jnp.min(jnp.where(logits == m, lane, _E), axis=-1, keepdims=True)
    wfc = wfc_ref[...]
    b2fc = jnp.dot(b2_ref[...], wfc, preferred_element_type=jnp.float32)
    e16 = e.astype(jnp.bfloat16)
    acc = jnp.zeros((e.shape[0], _OUT), jnp.float32)
    for ei in range(_E):
        h = jnp.maximum(
            jnp.dot(e16, w1_ref[ei].astype(jnp.bfloat16),
                    preferred_element_type=jnp.float32)
            + b1_ref[ei], 0.0)
        w2fc = jnp.dot(w2_ref[ei], wfc, preferred_element_type=jnp.float32)
        t = jnp.dot(h.astype(jnp.bfloat16), w2fc.astype(jnp.bfloat16),
                    preferred_element_type=jnp.float32) \
            + b2fc[ei:ei + 1, :]
        acc = acc + jnp.where(top1 == ei, t, 0.0)
    y = gate * acc + bfc_ref[...]
    my = jnp.max(y, axis=-1, keepdims=True)
    lse = my + jnp.log(jnp.sum(jnp.exp(y - my), axis=-1, keepdims=True))
    out_ref[...] = y - lse


def _moe(embedded, W_g, W1, b1, W2, b2, W_fc, b_fc, interpret=False):
    return pl.pallas_call(
        _moe_body,
        grid=(_B // _BLK,),
        in_specs=[
            pl.BlockSpec((_BLK, _D), lambda i: (i, 0)),
            pl.BlockSpec((_D, _E), lambda i: (0, 0)),
            pl.BlockSpec((_E, _D, _FFN), lambda i: (0, 0, 0)),
            pl.BlockSpec((_E, 1, _FFN), lambda i: (0, 0, 0)),
            pl.BlockSpec((_E, _FFN, _D), lambda i: (0, 0, 0)),
            pl.BlockSpec((_E, _D), lambda i: (0, 0)),
            pl.BlockSpec((_D, _OUT), lambda i: (0, 0)),
            pl.BlockSpec((1, _OUT), lambda i: (0, 0)),
        ],
        out_specs=pl.BlockSpec((_BLK, _OUT), lambda i: (i, 0)),
        out_shape=jax.ShapeDtypeStruct((_B, _OUT), jnp.float32),
        compiler_params=pltpu.CompilerParams(
            dimension_semantics=("arbitrary",)),
        interpret=interpret,
    )(embedded, W_g, W1, b1.reshape(_E, 1, _FFN), W2,
      b2, W_fc, b_fc.reshape(1, _OUT))


def kernel(x, emb, W_g, W1, b1, W2, b2, W_fc, b_fc):
    x_flat = x.reshape(-1).astype(jnp.int32)
    embedded = _pool()(x_flat, emb)
    return _moe(embedded, W_g, W1, b1, W2, b2, W_fc, b_fc)
